# baseline (speedup 1.0000x reference)
import jax
import jax.numpy as jnp
from jax import lax
from jax.experimental import pallas as pl
from jax.experimental.pallas import tpu as pltpu
from jax.experimental.pallas import tpu_sc as plsc

_DIM = 1024
_SCALE = _DIM ** (-0.5)
_ROWS = 8192
_NC = 2
_NS = 16
_NW = _NC * _NS
_WROWS = _ROWS // _NW
_CROWS = 8
_NBUF = 14
_NCHUNK = _WROWS // _CROWS
_CVECS = _CROWS * _DIM // 16


def _sc_body(emb_hbm, out_hbm, bufs, lsems, ssems):
    c = lax.axis_index("c")
    s = lax.axis_index("s")
    wid = s * _NC + c

    def ld(g):
        row = (g * _NW + wid) * _CROWS
        pltpu.async_copy(
            emb_hbm.at[pl.ds(row, _CROWS), :],
            bufs[g % _NBUF], lsems[g % _NBUF])

    def ld_wait(g):
        row = (g * _NW + wid) * _CROWS
        pltpu.make_async_copy(
            emb_hbm.at[pl.ds(row, _CROWS), :],
            bufs[g % _NBUF], lsems[g % _NBUF]).wait()

    def st(g):
        row = (g * _NW + wid) * _CROWS
        pltpu.async_copy(
            bufs[g % _NBUF],
            out_hbm.at[pl.ds(row, _CROWS), :],
            ssems[g % _NBUF])

    def st_wait(g):
        row = (g * _NW + wid) * _CROWS
        pltpu.make_async_copy(
            bufs[g % _NBUF],
            out_hbm.at[pl.ds(row, _CROWS), :],
            ssems[g % _NBUF]).wait()

    for g in range(_NBUF - 1):
        ld(g)
    for g in range(_NCHUNK):
        b = g % _NBUF
        ld_wait(g)
        if g + _NBUF - 1 < _NCHUNK:
            if g >= 1:
                st_wait(g - 1)
            ld(g + _NBUF - 1)

        @plsc.parallel_loop(0, _CVECS, 1, unroll=16)
        def _scale(i):
            r = i >> 6
            col = pl.multiple_of((i & 63) * 16, 16)
            bufs[b][r, pl.ds(col, 16)] = bufs[b][r, pl.ds(col, 16)] * _SCALE

        st(g)
    for g in range(_NCHUNK - _NBUF, _NCHUNK):
        st_wait(g)


_sc_scale = pl.kernel(
    _sc_body,
    out_type=jax.ShapeDtypeStruct((_ROWS, _DIM), jnp.float32),
    mesh=plsc.VectorSubcoreMesh(core_axis_name="c", subcore_axis_name="s"),
    scratch_types=[
        [pltpu.VMEM((_CROWS, _DIM), jnp.float32) for _ in range(_NBUF)],
        [pltpu.SemaphoreType.DMA for _ in range(_NBUF)],
        [pltpu.SemaphoreType.DMA for _ in range(_NBUF)],
    ],
)


def kernel(x, emb):
    seq_len = x.shape[1]
    return _sc_scale(emb[:seq_len])


# speedup vs baseline: 1.0802x; 1.0802x over previous
import jax
import jax.numpy as jnp
from jax import lax
from jax.experimental import pallas as pl
from jax.experimental.pallas import tpu as pltpu
from jax.experimental.pallas import tpu_sc as plsc

_DIM = 1024
_SCALE = _DIM ** (-0.5)
_ROWS = 8192
_NC = 2
_NS = 16
_NW = _NC * _NS
_WROWS = _ROWS // _NW
_CROWS = 16
_NBUF = 7
_NCHUNK = _WROWS // _CROWS
_CVECS = _CROWS * _DIM // 16


def _sc_body(emb_hbm, out_hbm, bufs, lsems, ssems):
    c = lax.axis_index("c")
    s = lax.axis_index("s")
    wid = s * _NC + c

    def ld(g):
        row = (g * _NW + wid) * _CROWS
        pltpu.async_copy(
            emb_hbm.at[pl.ds(row, _CROWS), :],
            bufs[g % _NBUF], lsems[g % _NBUF])

    def ld_wait(g):
        row = (g * _NW + wid) * _CROWS
        pltpu.make_async_copy(
            emb_hbm.at[pl.ds(row, _CROWS), :],
            bufs[g % _NBUF], lsems[g % _NBUF]).wait()

    def st(g):
        row = (g * _NW + wid) * _CROWS
        pltpu.async_copy(
            bufs[g % _NBUF],
            out_hbm.at[pl.ds(row, _CROWS), :],
            ssems[g % _NBUF])

    def st_wait(g):
        row = (g * _NW + wid) * _CROWS
        pltpu.make_async_copy(
            bufs[g % _NBUF],
            out_hbm.at[pl.ds(row, _CROWS), :],
            ssems[g % _NBUF]).wait()

    for g in range(_NBUF - 1):
        ld(g)
    for g in range(_NCHUNK):
        b = g % _NBUF
        ld_wait(g)
        if g + _NBUF - 1 < _NCHUNK:
            if g >= 1:
                st_wait(g - 1)
            ld(g + _NBUF - 1)

        @plsc.parallel_loop(0, _CVECS, 1, unroll=16)
        def _scale(i):
            r = i >> 6
            col = pl.multiple_of((i & 63) * 16, 16)
            bufs[b][r, pl.ds(col, 16)] = bufs[b][r, pl.ds(col, 16)] * _SCALE

        st(g)
    for g in range(_NCHUNK - _NBUF, _NCHUNK):
        st_wait(g)


_sc_scale = pl.kernel(
    _sc_body,
    out_type=jax.ShapeDtypeStruct((_ROWS, _DIM), jnp.float32),
    mesh=plsc.VectorSubcoreMesh(core_axis_name="c", subcore_axis_name="s"),
    scratch_types=[
        [pltpu.VMEM((_CROWS, _DIM), jnp.float32) for _ in range(_NBUF)],
        [pltpu.SemaphoreType.DMA for _ in range(_NBUF)],
        [pltpu.SemaphoreType.DMA for _ in range(_NBUF)],
    ],
)


def kernel(x, emb):
    seq_len = x.shape[1]
    return _sc_scale(emb[:seq_len])
